# baseline (speedup 1.0000x reference)
import functools

import jax
import jax.numpy as jnp
from jax import lax
from jax.experimental import pallas as pl
from jax.experimental.pallas import tpu as pltpu
from jax.experimental.pallas import tpu_sc as plsc

_MARK_AX = 0
_OP_SHL = 1
_OP_SHR = 2
_ALU_LO = 4
_ALU_HI = 20
_AX_CARRY_LO = 36
_OUTPUT_LO = 52
_OUTPUT_HI = 68

_CHUNK = 128

_DNUMS = lax.GatherDimensionNumbers(
    offset_dims=(), collapsed_slice_dims=(0,), start_index_map=(0,))


def _shuf(v, perm):
    return lax.gather(v, perm[:, None], _DNUMS, slice_sizes=(1,),
                      mode=lax.GatherScatterMode.PROMISE_IN_BOUNDS)


def kernel(x_bd, powers):
    del powers
    b, s, d = x_bd.shape
    n = b * s
    x = x_bd.reshape(n, d)

    info = plsc.get_sparse_core_info()
    nc, ns = info.num_cores, info.num_subcores
    nw = nc * ns
    tpw = n // nw
    c = min(_CHUNK, tpw)
    mesh = plsc.VectorSubcoreMesh(core_axis_name="c", subcore_axis_name="s")

    nchunks = tpw // c

    @functools.partial(
        pl.kernel,
        out_type=jax.ShapeDtypeStruct((n, d), jnp.float32),
        mesh=mesh,
        scratch_types=[
            pltpu.VMEM((c, d), jnp.float32),
            pltpu.VMEM((c, d), jnp.float32),
            pltpu.SemaphoreType.DMA,
            pltpu.SemaphoreType.DMA,
            pltpu.SemaphoreType.DMA,
            pltpu.SemaphoreType.DMA,
        ],
    )
    def sc_kernel(x_hbm, out_hbm, buf0, buf1, si0, si1, so0, so1):
        wid = lax.axis_index("s") * nc + lax.axis_index("c")
        base0 = wid * tpw
        bufs = (buf0, buf1)
        sin = (si0, si1)
        sout = (so0, so1)

        def start_load(g):
            return pltpu.async_copy(
                x_hbm.at[pl.ds(base0 + g * c, c)], bufs[g % 2], sin[g % 2])

        def start_store(g):
            return pltpu.async_copy(
                bufs[g % 2], out_hbm.at[pl.ds(base0 + g * c, c)], sout[g % 2])

        def compute(buf):

            @pl.loop(0, c // 2, unroll=2)
            def _pair_loop(kk):
                t_a = kk * 2
                t_b = t_a + 1
                iota = lax.iota(jnp.int32, 16)
                half_f = (iota & 7).astype(jnp.float32)
                mask_lo = iota < 8

                def argmax16_pair(offs):
                    a = buf[t_a, pl.ds(offs, 16)]
                    b2 = buf[t_b, pl.ds(offs, 16)]
                    p = jnp.where(mask_lo, a, _shuf(b2, iota ^ 8))
                    q = jnp.where(mask_lo, _shuf(a, iota ^ 8), b2)
                    r = jnp.maximum(p, q)
                    for sft in (1, 2, 4):
                        r = jnp.maximum(r, _shuf(r, iota ^ sft))
                    cand = jnp.minimum(
                        jnp.where(p == r, half_f, 16.0),
                        jnp.where(q == r, half_f + 8.0, 16.0))
                    for sft in (1, 2, 4):
                        cand = jnp.minimum(cand, _shuf(cand, iota ^ sft))
                    return cand

                val_lo = argmax16_pair(_ALU_LO)
                val_hi = argmax16_pair(_ALU_HI)
                sh_i = argmax16_pair(_AX_CARRY_LO).astype(jnp.int32)

                head_a = buf[t_a, pl.ds(0, 16)]
                head_b = buf[t_b, pl.ds(0, 16)]

                def flags(h):
                    mark_f = jnp.where(h[_MARK_AX] >= 0.5, 1.0, 0.0)
                    shl_f = jnp.where(h[_OP_SHL] > 0.5, 1.0, 0.0)
                    shr_f = jnp.where(h[_OP_SHR] > 0.5, 1.0, 0.0) * (
                        1.0 - shl_f)
                    return 2.0 * mark_f * (shl_f + shr_f), shl_f

                act2_a, shl_a = flags(head_a)
                act2_b, shl_b = flags(head_b)
                shl_half = jnp.where(mask_lo, shl_a, shl_b)

                value = val_lo + val_hi * 16.0
                power = lax.bitcast_convert_type((sh_i + 127) << 23,
                                                 jnp.float32)
                inv_power = lax.bitcast_convert_type((127 - sh_i) << 23,
                                                     jnp.float32)
                res_shl = jnp.bitwise_and((value * power).astype(jnp.int32),
                                          255)
                res_shr = (value * inv_power).astype(jnp.int32)
                result = jnp.where(shl_half > 0.5, res_shl, res_shr)

                res_a = _shuf(result, iota & 7)
                res_b = _shuf(result, (iota & 7) | 8)

                add_lo_a = jnp.where(iota == (res_a & 15), act2_a, 0.0)
                add_hi_a = jnp.where(iota == ((res_a >> 4) & 15), act2_a, 0.0)
                add_lo_b = jnp.where(iota == (res_b & 15), act2_b, 0.0)
                add_hi_b = jnp.where(iota == ((res_b >> 4) & 15), act2_b, 0.0)
                plsc.addupdate(buf.at[t_a, pl.ds(_OUTPUT_LO, 16)], add_lo_a)
                plsc.addupdate(buf.at[t_a, pl.ds(_OUTPUT_HI, 16)], add_hi_a)
                plsc.addupdate(buf.at[t_b, pl.ds(_OUTPUT_LO, 16)], add_lo_b)
                plsc.addupdate(buf.at[t_b, pl.ds(_OUTPUT_HI, 16)], add_hi_b)

        loads = {0: start_load(0)}
        stores = {}
        for g in range(nchunks):
            if g + 1 < nchunks:
                if g - 1 >= 0:
                    stores[g - 1].wait()
                loads[g + 1] = start_load(g + 1)
            loads[g].wait()
            compute(bufs[g % 2])
            stores[g] = start_store(g)
        for g in range(max(0, nchunks - 2), nchunks):
            stores[g].wait()

    return sc_kernel(x).reshape(b, s, d)


# speedup vs baseline: 1.0176x; 1.0176x over previous
import functools

import jax
import jax.numpy as jnp
from jax import lax
from jax.experimental import pallas as pl
from jax.experimental.pallas import tpu as pltpu
from jax.experimental.pallas import tpu_sc as plsc

_MARK_AX = 0
_OP_SHL = 1
_OP_SHR = 2
_ALU_LO = 4
_ALU_HI = 20
_AX_CARRY_LO = 36
_OUTPUT_LO = 52
_OUTPUT_HI = 68

_CHUNK = 256

_DNUMS = lax.GatherDimensionNumbers(
    offset_dims=(), collapsed_slice_dims=(0,), start_index_map=(0,))


def _shuf(v, perm):
    return lax.gather(v, perm[:, None], _DNUMS, slice_sizes=(1,),
                      mode=lax.GatherScatterMode.PROMISE_IN_BOUNDS)


def kernel(x_bd, powers):
    del powers
    b, s, d = x_bd.shape
    n = b * s
    x = x_bd.reshape(n, d)

    info = plsc.get_sparse_core_info()
    nc, ns = info.num_cores, info.num_subcores
    nw = nc * ns
    tpw = n // nw
    c = min(_CHUNK, tpw)
    mesh = plsc.VectorSubcoreMesh(core_axis_name="c", subcore_axis_name="s")

    nchunks = tpw // c

    @functools.partial(
        pl.kernel,
        out_type=jax.ShapeDtypeStruct((n, d), jnp.float32),
        mesh=mesh,
        scratch_types=[
            pltpu.VMEM((c, d), jnp.float32),
            pltpu.VMEM((c, d), jnp.float32),
            pltpu.SemaphoreType.DMA,
            pltpu.SemaphoreType.DMA,
            pltpu.SemaphoreType.DMA,
            pltpu.SemaphoreType.DMA,
        ],
    )
    def sc_kernel(x_hbm, out_hbm, buf0, buf1, si0, si1, so0, so1):
        wid = lax.axis_index("s") * nc + lax.axis_index("c")
        base0 = wid * tpw
        bufs = (buf0, buf1)
        sin = (si0, si1)
        sout = (so0, so1)

        def start_load(g):
            return pltpu.async_copy(
                x_hbm.at[pl.ds(base0 + g * c, c)], bufs[g % 2], sin[g % 2])

        def start_store(g):
            return pltpu.async_copy(
                bufs[g % 2], out_hbm.at[pl.ds(base0 + g * c, c)], sout[g % 2])

        def compute(buf):

            @pl.loop(0, c // 2, unroll=2)
            def _pair_loop(kk):
                t_a = kk * 2
                t_b = t_a + 1
                iota = lax.iota(jnp.int32, 16)
                half_f = (iota & 7).astype(jnp.float32)
                mask_lo = iota < 8

                def argmax16_pair(offs):
                    a = buf[t_a, pl.ds(offs, 16)]
                    b2 = buf[t_b, pl.ds(offs, 16)]
                    p = jnp.where(mask_lo, a, _shuf(b2, iota ^ 8))
                    q = jnp.where(mask_lo, _shuf(a, iota ^ 8), b2)
                    r = jnp.maximum(p, q)
                    for sft in (1, 2, 4):
                        r = jnp.maximum(r, _shuf(r, iota ^ sft))
                    cand = jnp.minimum(
                        jnp.where(p == r, half_f, 16.0),
                        jnp.where(q == r, half_f + 8.0, 16.0))
                    for sft in (1, 2, 4):
                        cand = jnp.minimum(cand, _shuf(cand, iota ^ sft))
                    return cand

                val_lo = argmax16_pair(_ALU_LO)
                val_hi = argmax16_pair(_ALU_HI)
                sh_i = argmax16_pair(_AX_CARRY_LO).astype(jnp.int32)

                head_a = buf[t_a, pl.ds(0, 16)]
                head_b = buf[t_b, pl.ds(0, 16)]

                def flags(h):
                    mark_f = jnp.where(h[_MARK_AX] >= 0.5, 1.0, 0.0)
                    shl_f = jnp.where(h[_OP_SHL] > 0.5, 1.0, 0.0)
                    shr_f = jnp.where(h[_OP_SHR] > 0.5, 1.0, 0.0) * (
                        1.0 - shl_f)
                    return 2.0 * mark_f * (shl_f + shr_f), shl_f

                act2_a, shl_a = flags(head_a)
                act2_b, shl_b = flags(head_b)
                shl_half = jnp.where(mask_lo, shl_a, shl_b)

                value = val_lo + val_hi * 16.0
                power = lax.bitcast_convert_type((sh_i + 127) << 23,
                                                 jnp.float32)
                inv_power = lax.bitcast_convert_type((127 - sh_i) << 23,
                                                     jnp.float32)
                res_shl = jnp.bitwise_and((value * power).astype(jnp.int32),
                                          255)
                res_shr = (value * inv_power).astype(jnp.int32)
                result = jnp.where(shl_half > 0.5, res_shl, res_shr)

                res_a = _shuf(result, iota & 7)
                res_b = _shuf(result, (iota & 7) | 8)

                add_lo_a = jnp.where(iota == (res_a & 15), act2_a, 0.0)
                add_hi_a = jnp.where(iota == ((res_a >> 4) & 15), act2_a, 0.0)
                add_lo_b = jnp.where(iota == (res_b & 15), act2_b, 0.0)
                add_hi_b = jnp.where(iota == ((res_b >> 4) & 15), act2_b, 0.0)
                plsc.addupdate(buf.at[t_a, pl.ds(_OUTPUT_LO, 16)], add_lo_a)
                plsc.addupdate(buf.at[t_a, pl.ds(_OUTPUT_HI, 16)], add_hi_a)
                plsc.addupdate(buf.at[t_b, pl.ds(_OUTPUT_LO, 16)], add_lo_b)
                plsc.addupdate(buf.at[t_b, pl.ds(_OUTPUT_HI, 16)], add_hi_b)

        loads = {0: start_load(0)}
        stores = {}
        for g in range(nchunks):
            if g + 1 < nchunks:
                if g - 1 >= 0:
                    stores[g - 1].wait()
                loads[g + 1] = start_load(g + 1)
            loads[g].wait()
            compute(bufs[g % 2])
            stores[g] = start_store(g)
        for g in range(max(0, nchunks - 2), nchunks):
            stores[g].wait()

    return sc_kernel(x).reshape(b, s, d)
